# Initial kernel scaffold; baseline (speedup 1.0000x reference)
import jax
import jax.numpy as jnp
from jax.experimental import pallas as pl


def kernel(index, hidden_states, experts_cache, gate_w, ws):
    raise NotImplementedError("write your pallas kernel here")


# speedup vs baseline: 2.3726x; 2.3726x over previous
import functools

import jax
import jax.numpy as jnp
from jax import lax
from jax.experimental import pallas as pl
from jax.experimental.pallas import tpu as pltpu

E = 8
H = 1024
I = 2048
BT = 128
NT = 40
ROWS = NT * BT
TBLK = 256
MM_DTYPE = jnp.bfloat16


def _router_body(x_ref, gw_ref, sel_ref, wv_ref):
    xb = x_ref[...]
    gw = gw_ref[...]
    l = lax.dot_general(xb, gw, (((1,), (1,)), ((), ())),
                        preferred_element_type=jnp.float32)
    m = jnp.max(l, axis=1, keepdims=True)
    q = jnp.exp(l - m)
    ii = lax.broadcasted_iota(jnp.int32, l.shape, 1)
    m1 = jnp.max(q, axis=1, keepdims=True)
    i1 = jnp.min(jnp.where(q == m1, ii, E), axis=1, keepdims=True)
    q2 = jnp.where(ii == i1, -1.0, q)
    m2 = jnp.max(q2, axis=1, keepdims=True)
    i2 = jnp.min(jnp.where(q2 == m2, ii, E), axis=1, keepdims=True)
    s = m1 + m2
    sel_ref[...] = jnp.where((ii == i1) | (ii == i2), 1.0, 0.0)
    wv_ref[...] = (jnp.where(ii == i1, m1 / s, 0.0)
                   + jnp.where(ii == i2, m2 / s, 0.0))


def _ffn_body(te_ref, nv_ref, ca_ref, xs_ref, w1_ref, w3_ref, w2_ref, ys_ref):
    i = pl.program_id(0)

    @pl.when(i < nv_ref[0])
    def _():
        xb = xs_ref[...].astype(MM_DTYPE)
        w1 = w1_ref[0].astype(MM_DTYPE)
        w3 = w3_ref[0].astype(MM_DTYPE)
        h1 = lax.dot_general(xb, w1, (((1,), (1,)), ((), ())),
                             preferred_element_type=jnp.float32)
        h3 = lax.dot_general(xb, w3, (((1,), (1,)), ((), ())),
                             preferred_element_type=jnp.float32)
        act = (h1 * lax.logistic(h1) * h3).astype(MM_DTYPE)
        w2 = w2_ref[0].astype(MM_DTYPE)
        ys_ref[...] = lax.dot_general(act, w2, (((1,), (1,)), ((), ())),
                                      preferred_element_type=jnp.float32)


def kernel(index, hidden_states, experts_cache, gate_w, ws):
    x = hidden_states
    T = x.shape[0]
    gw = gate_w[index]

    sel, wv = pl.pallas_call(
        _router_body,
        grid=(T // TBLK,),
        in_specs=[pl.BlockSpec((TBLK, H), lambda i: (i, 0)),
                  pl.BlockSpec((E, H), lambda i: (0, 0))],
        out_specs=[pl.BlockSpec((TBLK, E), lambda i: (i, 0)),
                   pl.BlockSpec((TBLK, E), lambda i: (i, 0))],
        out_shape=[jax.ShapeDtypeStruct((T, E), jnp.float32),
                   jax.ShapeDtypeStruct((T, E), jnp.float32)],
    )(x, gw)

    selb = sel > 0.5
    seli = selb.astype(jnp.int32)
    cnt = jnp.sum(seli, axis=0)
    pos = jnp.cumsum(seli, axis=0) - seli
    tiles = (cnt + BT - 1) // BT
    offt = (jnp.cumsum(tiles) - tiles).astype(jnp.int32)
    nvalid = jnp.sum(tiles).astype(jnp.int32)
    dest = offt[None, :] * BT + pos
    tok = lax.broadcasted_iota(jnp.int32, (T, E), 0)
    flat_dest = jnp.where(selb, dest, ROWS)
    src_rows = (jnp.zeros((ROWS + 1,), jnp.int32)
                .at[flat_dest.reshape(-1)].set(tok.reshape(-1))[:ROWS])
    ord_ = jnp.argsort(1 - seli, axis=1, stable=True)
    e0 = ord_[:, 0]
    e1 = ord_[:, 1]
    ar = jnp.arange(T)
    r0 = dest[ar, e0]
    r1 = dest[ar, e1]
    w0 = wv[ar, e0]
    w1_ = wv[ar, e1]
    ti = jnp.arange(NT, dtype=jnp.int32)
    tile_e = jnp.clip(jnp.searchsorted(offt, ti, side='right') - 1,
                      0, E - 1).astype(jnp.int32)

    xs = jnp.take(x, src_rows, axis=0)
    ws0 = ws[index]
    wa = ws0.reshape(E, 3 * I, H)
    wb = ws0.reshape(E, 3 * H, I)
    cache = experts_cache.astype(jnp.int32)

    grid_spec = pltpu.PrefetchScalarGridSpec(
        num_scalar_prefetch=3,
        grid=(NT,),
        in_specs=[
            pl.BlockSpec((BT, H), lambda i, te, nv, ca: (i, 0)),
            pl.BlockSpec((1, I, H), lambda i, te, nv, ca: (ca[te[i]], 0, 0)),
            pl.BlockSpec((1, I, H), lambda i, te, nv, ca: (ca[te[i]], 1, 0)),
            pl.BlockSpec((1, H, I), lambda i, te, nv, ca: (ca[te[i]], 2, 0)),
        ],
        out_specs=pl.BlockSpec((BT, H), lambda i, te, nv, ca: (i, 0)),
    )
    ys = pl.pallas_call(
        _ffn_body,
        grid_spec=grid_spec,
        out_shape=jax.ShapeDtypeStruct((ROWS, H), jnp.float32),
        compiler_params=pltpu.CompilerParams(
            dimension_semantics=("arbitrary",)),
    )(tile_e, jnp.reshape(nvalid, (1,)), cache, xs, wa, wa, wb)

    out = w0[:, None] * ys[r0] + w1_[:, None] * ys[r1]
    return out


# speedup vs baseline: 2.3796x; 1.0030x over previous
import functools

import jax
import jax.numpy as jnp
from jax import lax
from jax.experimental import pallas as pl
from jax.experimental.pallas import tpu as pltpu

E = 8
H = 1024
I = 2048
BT = 128
NT = 40
ROWS = NT * BT
TBLK = 256
MM_DTYPE = jnp.bfloat16


def _router_body(x_ref, gw_ref, sel_ref, wv_ref):
    xb = x_ref[...]
    gw = gw_ref[...]
    l = lax.dot_general(xb, gw, (((1,), (1,)), ((), ())),
                        preferred_element_type=jnp.float32)
    m = jnp.max(l, axis=1, keepdims=True)
    q = jnp.exp(l - m)
    ii = lax.broadcasted_iota(jnp.int32, l.shape, 1)
    m1 = jnp.max(q, axis=1, keepdims=True)
    i1 = jnp.min(jnp.where(q == m1, ii, E), axis=1, keepdims=True)
    q2 = jnp.where(ii == i1, -1.0, q)
    m2 = jnp.max(q2, axis=1, keepdims=True)
    i2 = jnp.min(jnp.where(q2 == m2, ii, E), axis=1, keepdims=True)
    s = m1 + m2
    sel_ref[...] = jnp.where((ii == i1) | (ii == i2), 1.0, 0.0)
    wv_ref[...] = (jnp.where(ii == i1, m1 / s, 0.0)
                   + jnp.where(ii == i2, m2 / s, 0.0))


def _ffn_body(te_ref, nv_ref, ca_ref, xs_ref, w1_ref, w3_ref, w2_ref, ys_ref):
    i = pl.program_id(0)

    @pl.when(i < nv_ref[0])
    def _():
        xb = xs_ref[...].astype(MM_DTYPE)
        w1 = w1_ref[0].astype(MM_DTYPE)
        w3 = w3_ref[0].astype(MM_DTYPE)
        h1 = lax.dot_general(xb, w1, (((1,), (1,)), ((), ())),
                             preferred_element_type=jnp.float32)
        h3 = lax.dot_general(xb, w3, (((1,), (1,)), ((), ())),
                             preferred_element_type=jnp.float32)
        act = (h1 * lax.logistic(h1) * h3).astype(MM_DTYPE)
        w2 = w2_ref[0].astype(MM_DTYPE)
        ys_ref[...] = lax.dot_general(act, w2, (((1,), (1,)), ((), ())),
                                      preferred_element_type=jnp.float32)


def kernel(index, hidden_states, experts_cache, gate_w, ws):
    x = hidden_states
    T = x.shape[0]
    del index
    gw = gate_w.reshape(E, H)

    sel, wv = pl.pallas_call(
        _router_body,
        grid=(T // TBLK,),
        in_specs=[pl.BlockSpec((TBLK, H), lambda i: (i, 0)),
                  pl.BlockSpec((E, H), lambda i: (0, 0))],
        out_specs=[pl.BlockSpec((TBLK, E), lambda i: (i, 0)),
                   pl.BlockSpec((TBLK, E), lambda i: (i, 0))],
        out_shape=[jax.ShapeDtypeStruct((T, E), jnp.float32),
                   jax.ShapeDtypeStruct((T, E), jnp.float32)],
    )(x, gw)

    selb = sel > 0.5
    seli = selb.astype(jnp.int32)
    cnt = jnp.sum(seli, axis=0)
    pos = jnp.cumsum(seli, axis=0) - seli
    tiles = (cnt + BT - 1) // BT
    offt = (jnp.cumsum(tiles) - tiles).astype(jnp.int32)
    nvalid = jnp.sum(tiles).astype(jnp.int32)
    dest = offt[None, :] * BT + pos
    tok = lax.broadcasted_iota(jnp.int32, (T, E), 0)
    flat_dest = jnp.where(selb, dest, ROWS)
    src_rows = (jnp.zeros((ROWS + 1,), jnp.int32)
                .at[flat_dest.reshape(-1)].set(tok.reshape(-1))[:ROWS])
    ord_ = jnp.argsort(1 - seli, axis=1, stable=True)
    e0 = ord_[:, 0]
    e1 = ord_[:, 1]
    ar = jnp.arange(T)
    r0 = dest[ar, e0]
    r1 = dest[ar, e1]
    w0 = wv[ar, e0]
    w1_ = wv[ar, e1]
    ti = jnp.arange(NT, dtype=jnp.int32)
    tile_e = jnp.clip(jnp.searchsorted(offt, ti, side='right') - 1,
                      0, E - 1).astype(jnp.int32)

    xs = jnp.take(x, src_rows, axis=0)
    wa = ws.reshape(E, 3 * I, H)
    wb = ws.reshape(E, 3 * H, I)
    cache = experts_cache.astype(jnp.int32)

    grid_spec = pltpu.PrefetchScalarGridSpec(
        num_scalar_prefetch=3,
        grid=(NT,),
        in_specs=[
            pl.BlockSpec((BT, H), lambda i, te, nv, ca: (i, 0)),
            pl.BlockSpec((1, I, H), lambda i, te, nv, ca: (ca[te[i]], 0, 0)),
            pl.BlockSpec((1, I, H), lambda i, te, nv, ca: (ca[te[i]], 1, 0)),
            pl.BlockSpec((1, H, I), lambda i, te, nv, ca: (ca[te[i]], 2, 0)),
        ],
        out_specs=pl.BlockSpec((BT, H), lambda i, te, nv, ca: (i, 0)),
    )
    ys = pl.pallas_call(
        _ffn_body,
        grid_spec=grid_spec,
        out_shape=jax.ShapeDtypeStruct((ROWS, H), jnp.float32),
        compiler_params=pltpu.CompilerParams(
            dimension_semantics=("arbitrary",)),
    )(tile_e, jnp.reshape(nvalid, (1,)), cache, xs, wa, wa, wb)

    out = w0[:, None] * ys[r0] + w1_[:, None] * ys[r1]
    return out


# speedup vs baseline: 2.7321x; 1.1481x over previous
import functools

import jax
import jax.numpy as jnp
from jax import lax
from jax.experimental import pallas as pl
from jax.experimental.pallas import tpu as pltpu

E = 8
H = 1024
I = 2048
BT = 128
NT = 40
ROWS = NT * BT
TBLK = 256
MM_DTYPE = jnp.bfloat16


def _router_body(x_ref, gw_ref, sel_ref, wv_ref):
    xb = x_ref[...]
    gw = gw_ref[...]
    l = lax.dot_general(xb, gw, (((1,), (1,)), ((), ())),
                        preferred_element_type=jnp.float32)
    m = jnp.max(l, axis=1, keepdims=True)
    q = jnp.exp(l - m)
    ii = lax.broadcasted_iota(jnp.int32, l.shape, 1)
    m1 = jnp.max(q, axis=1, keepdims=True)
    i1 = jnp.min(jnp.where(q == m1, ii, E), axis=1, keepdims=True)
    q2 = jnp.where(ii == i1, -1.0, q)
    m2 = jnp.max(q2, axis=1, keepdims=True)
    i2 = jnp.min(jnp.where(q2 == m2, ii, E), axis=1, keepdims=True)
    s = m1 + m2
    sel_ref[...] = jnp.where((ii == i1) | (ii == i2), 1.0, 0.0)
    wv_ref[...] = (jnp.where(ii == i1, m1 / s, 0.0)
                   + jnp.where(ii == i2, m2 / s, 0.0))


def _ffn_body(te_ref, nv_ref, ca_ref, xs_ref, w1_ref, w3_ref, w2_ref, ys_ref):
    i = pl.program_id(0)

    @pl.when(i < nv_ref[0])
    def _():
        xb = xs_ref[...].astype(MM_DTYPE)
        w1 = w1_ref[0]
        w3 = w3_ref[0]
        h1 = lax.dot_general(xb, w1, (((1,), (1,)), ((), ())),
                             preferred_element_type=jnp.float32)
        h3 = lax.dot_general(xb, w3, (((1,), (1,)), ((), ())),
                             preferred_element_type=jnp.float32)
        act = (h1 * lax.logistic(h1) * h3).astype(MM_DTYPE)
        w2 = w2_ref[0]
        ys_ref[...] = lax.dot_general(act, w2, (((1,), (1,)), ((), ())),
                                      preferred_element_type=jnp.float32)


def kernel(index, hidden_states, experts_cache, gate_w, ws):
    x = hidden_states
    T = x.shape[0]
    del index
    gw = gate_w.reshape(E, H)

    sel, wv = pl.pallas_call(
        _router_body,
        grid=(T // TBLK,),
        in_specs=[pl.BlockSpec((TBLK, H), lambda i: (i, 0)),
                  pl.BlockSpec((E, H), lambda i: (0, 0))],
        out_specs=[pl.BlockSpec((TBLK, E), lambda i: (i, 0)),
                   pl.BlockSpec((TBLK, E), lambda i: (i, 0))],
        out_shape=[jax.ShapeDtypeStruct((T, E), jnp.float32),
                   jax.ShapeDtypeStruct((T, E), jnp.float32)],
    )(x, gw)

    selb = sel > 0.5
    seli = selb.astype(jnp.int32)
    cnt = jnp.sum(seli, axis=0)
    pos = jnp.cumsum(seli, axis=0) - seli
    tiles = (cnt + BT - 1) // BT
    offt = (jnp.cumsum(tiles) - tiles).astype(jnp.int32)
    nvalid = jnp.sum(tiles).astype(jnp.int32)
    dest = offt[None, :] * BT + pos
    tok = lax.broadcasted_iota(jnp.int32, (T, E), 0)
    flat_dest = jnp.where(selb, dest, ROWS)
    src_rows = (jnp.zeros((ROWS + 1,), jnp.int32)
                .at[flat_dest.reshape(-1)].set(tok.reshape(-1))[:ROWS])
    ord_ = jnp.argsort(1 - seli, axis=1, stable=True)
    e0 = ord_[:, 0]
    e1 = ord_[:, 1]
    ar = jnp.arange(T)
    r0 = dest[ar, e0]
    r1 = dest[ar, e1]
    w0 = wv[ar, e0]
    w1_ = wv[ar, e1]
    ti = jnp.arange(NT, dtype=jnp.int32)
    tile_e = jnp.clip(jnp.searchsorted(offt, ti, side='right') - 1,
                      0, E - 1).astype(jnp.int32)

    xs = jnp.take(x, src_rows, axis=0)
    wa = ws.reshape(E, 3 * I, H).astype(MM_DTYPE)
    wb = ws.reshape(E, 3 * H, I).astype(MM_DTYPE)
    cache = experts_cache.astype(jnp.int32)

    grid_spec = pltpu.PrefetchScalarGridSpec(
        num_scalar_prefetch=3,
        grid=(NT,),
        in_specs=[
            pl.BlockSpec((BT, H), lambda i, te, nv, ca: (i, 0)),
            pl.BlockSpec((1, I, H), lambda i, te, nv, ca: (ca[te[i]], 0, 0)),
            pl.BlockSpec((1, I, H), lambda i, te, nv, ca: (ca[te[i]], 1, 0)),
            pl.BlockSpec((1, H, I), lambda i, te, nv, ca: (ca[te[i]], 2, 0)),
        ],
        out_specs=pl.BlockSpec((BT, H), lambda i, te, nv, ca: (i, 0)),
    )
    ys = pl.pallas_call(
        _ffn_body,
        grid_spec=grid_spec,
        out_shape=jax.ShapeDtypeStruct((ROWS, H), jnp.float32),
        compiler_params=pltpu.CompilerParams(
            dimension_semantics=("arbitrary",)),
    )(tile_e, jnp.reshape(nvalid, (1,)), cache, xs, wa, wa, wb)

    out = w0[:, None] * ys[r0] + w1_[:, None] * ys[r1]
    return out


# speedup vs baseline: 2.8976x; 1.0606x over previous
import functools

import jax
import jax.numpy as jnp
from jax import lax
from jax.experimental import pallas as pl
from jax.experimental.pallas import tpu as pltpu

E = 8
H = 1024
I = 2048
BT = 256
NT = 24
ROWS = NT * BT
TBLK = 256
MM_DTYPE = jnp.bfloat16


def _router_body(x_ref, gw_ref, sel_ref, wv_ref):
    xb = x_ref[...]
    gw = gw_ref[...]
    l = lax.dot_general(xb, gw, (((1,), (1,)), ((), ())),
                        preferred_element_type=jnp.float32)
    m = jnp.max(l, axis=1, keepdims=True)
    q = jnp.exp(l - m)
    ii = lax.broadcasted_iota(jnp.int32, l.shape, 1)
    m1 = jnp.max(q, axis=1, keepdims=True)
    i1 = jnp.min(jnp.where(q == m1, ii, E), axis=1, keepdims=True)
    q2 = jnp.where(ii == i1, -1.0, q)
    m2 = jnp.max(q2, axis=1, keepdims=True)
    i2 = jnp.min(jnp.where(q2 == m2, ii, E), axis=1, keepdims=True)
    s = m1 + m2
    sel_ref[...] = jnp.where((ii == i1) | (ii == i2), 1.0, 0.0)
    wv_ref[...] = (jnp.where(ii == i1, m1 / s, 0.0)
                   + jnp.where(ii == i2, m2 / s, 0.0))


def _ffn_body(te_ref, nv_ref, ca_ref, xs_ref, w1_ref, w3_ref, w2_ref, ys_ref):
    i = pl.program_id(0)

    @pl.when(i < nv_ref[0])
    def _():
        xb = xs_ref[...]
        w1 = w1_ref[0]
        w3 = w3_ref[0]
        h1 = lax.dot_general(xb, w1, (((1,), (1,)), ((), ())),
                             preferred_element_type=jnp.float32)
        h3 = lax.dot_general(xb, w3, (((1,), (1,)), ((), ())),
                             preferred_element_type=jnp.float32)
        act = (h1 * lax.logistic(h1) * h3).astype(MM_DTYPE)
        w2 = w2_ref[0]
        ys_ref[...] = lax.dot_general(act, w2, (((1,), (1,)), ((), ())),
                                      preferred_element_type=jnp.float32)


def kernel(index, hidden_states, experts_cache, gate_w, ws):
    x = hidden_states
    T = x.shape[0]
    del index
    gw = gate_w.reshape(E, H)

    sel, wv = pl.pallas_call(
        _router_body,
        grid=(T // TBLK,),
        in_specs=[pl.BlockSpec((TBLK, H), lambda i: (i, 0)),
                  pl.BlockSpec((E, H), lambda i: (0, 0))],
        out_specs=[pl.BlockSpec((TBLK, E), lambda i: (i, 0)),
                   pl.BlockSpec((TBLK, E), lambda i: (i, 0))],
        out_shape=[jax.ShapeDtypeStruct((T, E), jnp.float32),
                   jax.ShapeDtypeStruct((T, E), jnp.float32)],
    )(x, gw)

    selb = sel > 0.5
    seli = selb.astype(jnp.int32)
    cnt = jnp.sum(seli, axis=0)
    pos = jnp.cumsum(seli, axis=0) - seli
    tiles = (cnt + BT - 1) // BT
    offt = (jnp.cumsum(tiles) - tiles).astype(jnp.int32)
    nvalid = jnp.sum(tiles).astype(jnp.int32)
    dest = offt[None, :] * BT + pos
    tok = lax.broadcasted_iota(jnp.int32, (T, E), 0)
    flat_dest = jnp.where(selb, dest, ROWS)
    src_rows = (jnp.zeros((ROWS + 1,), jnp.int32)
                .at[flat_dest.reshape(-1)].set(tok.reshape(-1))[:ROWS])
    ord_ = jnp.argsort(1 - seli, axis=1, stable=True)
    e0 = ord_[:, 0]
    e1 = ord_[:, 1]
    ar = jnp.arange(T)
    r0 = dest[ar, e0]
    r1 = dest[ar, e1]
    w0 = wv[ar, e0]
    w1_ = wv[ar, e1]
    ti = jnp.arange(NT, dtype=jnp.int32)
    tile_e = jnp.clip(jnp.searchsorted(offt, ti, side='right') - 1,
                      0, E - 1).astype(jnp.int32)

    xs = jnp.take(x.astype(MM_DTYPE), src_rows, axis=0)
    wa = ws.reshape(E, 3 * I, H)[:, :2 * I].astype(MM_DTYPE)
    wb = ws.reshape(E, 3 * H, I)[:, 2 * H:].astype(MM_DTYPE)
    cache = experts_cache.astype(jnp.int32)

    grid_spec = pltpu.PrefetchScalarGridSpec(
        num_scalar_prefetch=3,
        grid=(NT,),
        in_specs=[
            pl.BlockSpec((BT, H), lambda i, te, nv, ca: (i, 0)),
            pl.BlockSpec((1, I, H), lambda i, te, nv, ca: (ca[te[i]], 0, 0)),
            pl.BlockSpec((1, I, H), lambda i, te, nv, ca: (ca[te[i]], 1, 0)),
            pl.BlockSpec((1, H, I), lambda i, te, nv, ca: (ca[te[i]], 0, 0)),
        ],
        out_specs=pl.BlockSpec((BT, H), lambda i, te, nv, ca: (i, 0)),
    )
    ys = pl.pallas_call(
        _ffn_body,
        grid_spec=grid_spec,
        out_shape=jax.ShapeDtypeStruct((ROWS, H), jnp.float32),
        compiler_params=pltpu.CompilerParams(
            dimension_semantics=("arbitrary",)),
    )(tile_e, jnp.reshape(nvalid, (1,)), cache, xs, wa, wa, wb)

    out = w0[:, None] * ys[r0] + w1_[:, None] * ys[r1]
    return out
